# baseline (speedup 1.0000x reference)
import jax
import jax.numpy as jnp
from jax.experimental import pallas as pl
from jax.experimental.pallas import tpu as pltpu

_CONF_THRESH = 0.25
_IOU_THRESH = 0.45
_MAX_PER_IMG = 100
_N = 5000
_NC = 80
_R = 40
_L = 128
_NP = _R * _L
_NEG = float("-inf")
_K = 4


_SX1, _SY1, _SX2, _SY2, _SAREA, _SOBJ, _SCCF, _SCPRED = range(8)


def _setup_one(x_ref, scr_ref, k):
    cx = x_ref[k, 0]
    cy = x_ref[k, 1]
    w = x_ref[k, 2]
    h = x_ref[k, 3]
    obj = x_ref[k, 4]

    x1 = cx - w / 2.0
    y1 = cy - h / 2.0
    x2 = cx + w / 2.0
    y2 = cy + h / 2.0
    area = (x2 - x1) * (y2 - y1)

    cls = x_ref[k, 5:5 + _NC]
    ccf = jnp.max(cls, axis=0)
    cidx = jax.lax.broadcasted_iota(jnp.int32, (_NC, _R, _L), 0)
    cpred = jnp.min(jnp.where(cls == ccf[None], cidx, 2**30), axis=0).astype(jnp.float32)

    score = obj * ccf
    ri = jax.lax.broadcasted_iota(jnp.int32, (_R, _L), 0)
    li = jax.lax.broadcasted_iota(jnp.int32, (_R, _L), 1)
    flat = ri * _L + li
    valid = flat < _N

    m0 = jnp.max(jnp.where(valid, score, _NEG), keepdims=True)
    conf = jnp.minimum(_CONF_THRESH, m0)
    s0 = jnp.where(valid & (score >= conf), score, _NEG)

    scr_ref[k, _SX1] = x1
    scr_ref[k, _SY1] = y1
    scr_ref[k, _SX2] = x2
    scr_ref[k, _SY2] = y2
    scr_ref[k, _SAREA] = area
    scr_ref[k, _SOBJ] = obj
    scr_ref[k, _SCCF] = ccf
    scr_ref[k, _SCPRED] = cpred
    return s0


def _nms_body(x_ref, o_ref, scr_ref):
    s0s = [_setup_one(x_ref, scr_ref, k) for k in range(_K)]
    lane8 = jax.lax.broadcasted_iota(jnp.int32, (1, 8), 1)
    row_iota = jax.lax.broadcasted_iota(jnp.int32, (_MAX_PER_IMG, 8), 0)
    ri = jax.lax.broadcasted_iota(jnp.int32, (_R, _L), 0)
    li = jax.lax.broadcasted_iota(jnp.int32, (_R, _L), 1)
    flat = ri * _L + li
    det0s = [jnp.zeros((_MAX_PER_IMG, 8), jnp.float32) for _ in range(_K)]

    def step(t, ss):
        out = []
        dets_out = []
        for k in range(_K):
            s = ss[k]
            dets = ss[_K + k]
            x1 = scr_ref[k, _SX1]
            y1 = scr_ref[k, _SY1]
            x2 = scr_ref[k, _SX2]
            y2 = scr_ref[k, _SY2]
            area = scr_ref[k, _SAREA]
            cpred = scr_ref[k, _SCPRED]
            m = jnp.max(s, keepdims=True)
            i = jnp.min(jnp.where(s == m, flat, 2**30), keepdims=True)
            pick = flat == i
            pf = pick.astype(jnp.float32)
            bx1 = jnp.sum(pf * x1, keepdims=True)
            by1 = jnp.sum(pf * y1, keepdims=True)
            bx2 = jnp.sum(pf * x2, keepdims=True)
            by2 = jnp.sum(pf * y2, keepdims=True)
            bobj = jnp.sum(pf * scr_ref[k, _SOBJ], keepdims=True)
            bccf = jnp.sum(pf * scr_ref[k, _SCCF], keepdims=True)
            bcls = jnp.sum(pf * cpred, keepdims=True)

            okf = jnp.where(m > _NEG, 1.0, 0.0)
            row = (jnp.where(lane8 == 0, bx1, 0.0)
                   + jnp.where(lane8 == 1, by1, 0.0)
                   + jnp.where(lane8 == 2, bx2, 0.0)
                   + jnp.where(lane8 == 3, by2, 0.0)
                   + jnp.where(lane8 == 4, bobj, 0.0)
                   + jnp.where(lane8 == 5, bccf, 0.0)
                   + jnp.where(lane8 == 6, bcls, 0.0)) * okf
            dets_out.append(dets + jnp.where(row_iota == t, 1.0, 0.0) * row)

            xx1 = jnp.maximum(bx1, x1)
            yy1 = jnp.maximum(by1, y1)
            xx2 = jnp.minimum(bx2, x2)
            yy2 = jnp.minimum(by2, y2)
            inter = jnp.maximum(xx2 - xx1, 0.0) * jnp.maximum(yy2 - yy1, 0.0)
            ba = (bx2 - bx1) * (by2 - by1)
            iou = inter / (ba + area - inter + 1e-9)
            sup = (iou > _IOU_THRESH) & (cpred == bcls)
            out.append(jnp.where(sup | pick, _NEG, s))
        return tuple(out + dets_out)

    fin = jax.lax.fori_loop(0, _MAX_PER_IMG, step, tuple(s0s + det0s))
    for k in range(_K):
        o_ref[k] = fin[_K + k]


def kernel(x):
    b, n, c = x.shape
    xp = jnp.pad(x, ((0, 0), (0, _NP - n), (0, 0)))
    xt = xp.transpose(0, 2, 1).reshape(b, c, _R, _L)
    out = pl.pallas_call(
        _nms_body,
        grid=(b // _K,),
        in_specs=[pl.BlockSpec((_K, c, _R, _L), lambda i: (i, 0, 0, 0))],
        out_specs=pl.BlockSpec((_K, _MAX_PER_IMG, 8), lambda i: (i, 0, 0)),
        out_shape=jax.ShapeDtypeStruct((b, _MAX_PER_IMG, 8), jnp.float32),
        scratch_shapes=[pltpu.VMEM((_K, 8, _R, _L), jnp.float32)],
        compiler_params=pltpu.CompilerParams(dimension_semantics=("parallel",)),
    )(xt)
    return out[:, :, :7]


# speedup vs baseline: 1.0234x; 1.0234x over previous
import jax
import jax.numpy as jnp
from jax.experimental import pallas as pl
from jax.experimental.pallas import tpu as pltpu

_CONF_THRESH = 0.25
_IOU_THRESH = 0.45
_MAX_PER_IMG = 100
_N = 5000
_NC = 80
_R = 40
_L = 128
_NP = _R * _L
_NEG = float("-inf")
_K = 4


_SX1, _SY1, _SX2, _SY2, _SAREA, _SOBJ, _SCCF, _SCPRED = range(8)


def _setup_one(x_ref, scr_ref, k):
    cx = x_ref[k, 0]
    cy = x_ref[k, 1]
    w = x_ref[k, 2]
    h = x_ref[k, 3]
    obj = x_ref[k, 4]

    x1 = cx - w / 2.0
    y1 = cy - h / 2.0
    x2 = cx + w / 2.0
    y2 = cy + h / 2.0
    area = (x2 - x1) * (y2 - y1)

    cls = x_ref[k, 5:5 + _NC]
    ccf = jnp.max(cls, axis=0)
    cidx = jax.lax.broadcasted_iota(jnp.int32, (_NC, _R, _L), 0)
    cpred = jnp.min(jnp.where(cls == ccf[None], cidx, 2**30), axis=0).astype(jnp.float32)

    score = obj * ccf
    ri = jax.lax.broadcasted_iota(jnp.int32, (_R, _L), 0)
    li = jax.lax.broadcasted_iota(jnp.int32, (_R, _L), 1)
    flat = ri * _L + li
    valid = flat < _N

    m0 = jnp.max(jnp.where(valid, score, _NEG), keepdims=True)
    conf = jnp.minimum(_CONF_THRESH, m0)
    s0 = jnp.where(valid & (score >= conf), score, _NEG)

    for cp in range(2):
        scr_ref[k, cp, _SX1] = x1
        scr_ref[k, cp, _SY1] = y1
        scr_ref[k, cp, _SX2] = x2
        scr_ref[k, cp, _SY2] = y2
        scr_ref[k, cp, _SAREA] = area
        scr_ref[k, cp, _SOBJ] = obj
        scr_ref[k, cp, _SCCF] = ccf
        scr_ref[k, cp, _SCPRED] = cpred
    return s0


def _nms_body(x_ref, o_ref, scr_ref):
    s0s = [_setup_one(x_ref, scr_ref, k) for k in range(_K)]
    lane8 = jax.lax.broadcasted_iota(jnp.int32, (1, 8), 1)

    def step(t, ss):
        cp = jax.lax.rem(t, 2)
        ri = jax.lax.broadcasted_iota(jnp.int32, (_R, _L), 0)
        li = jax.lax.broadcasted_iota(jnp.int32, (_R, _L), 1)
        flat = ri * _L + li
        out = []
        for k in range(_K):
            s = ss[k]
            x1 = scr_ref[k, cp, _SX1]
            y1 = scr_ref[k, cp, _SY1]
            x2 = scr_ref[k, cp, _SX2]
            y2 = scr_ref[k, cp, _SY2]
            area = scr_ref[k, cp, _SAREA]
            cpred = scr_ref[k, cp, _SCPRED]
            m = jnp.max(s, keepdims=True)
            i = jnp.min(jnp.where(s == m, flat, 2**30), keepdims=True)
            pick = flat == i
            pf = pick.astype(jnp.float32)
            bx1 = jnp.sum(pf * x1, keepdims=True)
            by1 = jnp.sum(pf * y1, keepdims=True)
            bx2 = jnp.sum(pf * x2, keepdims=True)
            by2 = jnp.sum(pf * y2, keepdims=True)
            bobj = jnp.sum(pf * scr_ref[k, cp, _SOBJ], keepdims=True)
            bccf = jnp.sum(pf * scr_ref[k, cp, _SCCF], keepdims=True)
            bcls = jnp.sum(pf * cpred, keepdims=True)

            okf = jnp.where(m > _NEG, 1.0, 0.0)
            row = (jnp.where(lane8 == 0, bx1, 0.0)
                   + jnp.where(lane8 == 1, by1, 0.0)
                   + jnp.where(lane8 == 2, bx2, 0.0)
                   + jnp.where(lane8 == 3, by2, 0.0)
                   + jnp.where(lane8 == 4, bobj, 0.0)
                   + jnp.where(lane8 == 5, bccf, 0.0)
                   + jnp.where(lane8 == 6, bcls, 0.0)) * okf
            o_ref[k, pl.ds(t, 1), :] = row

            xx1 = jnp.maximum(bx1, x1)
            yy1 = jnp.maximum(by1, y1)
            xx2 = jnp.minimum(bx2, x2)
            yy2 = jnp.minimum(by2, y2)
            inter = jnp.maximum(xx2 - xx1, 0.0) * jnp.maximum(yy2 - yy1, 0.0)
            ba = (bx2 - bx1) * (by2 - by1)
            iou = inter / (ba + area - inter + 1e-9)
            sup = (iou > _IOU_THRESH) & (cpred == bcls)
            out.append(jnp.where(sup | pick, _NEG, s))
        return tuple(out)

    jax.lax.fori_loop(0, _MAX_PER_IMG, step, tuple(s0s))


def kernel(x):
    b, n, c = x.shape
    xp = jnp.pad(x, ((0, 0), (0, _NP - n), (0, 0)))
    xt = xp.transpose(0, 2, 1).reshape(b, c, _R, _L)
    out = pl.pallas_call(
        _nms_body,
        grid=(b // _K,),
        in_specs=[pl.BlockSpec((_K, c, _R, _L), lambda i: (i, 0, 0, 0))],
        out_specs=pl.BlockSpec((_K, _MAX_PER_IMG, 8), lambda i: (i, 0, 0)),
        out_shape=jax.ShapeDtypeStruct((b, _MAX_PER_IMG, 8), jnp.float32),
        scratch_shapes=[pltpu.VMEM((_K, 2, 8, _R, _L), jnp.float32)],
        compiler_params=pltpu.CompilerParams(dimension_semantics=("parallel",)),
    )(xt)
    return out[:, :, :7]


# speedup vs baseline: 1.6614x; 1.6234x over previous
import jax
import jax.numpy as jnp
from jax.experimental import pallas as pl
from jax.experimental.pallas import tpu as pltpu

_CONF_THRESH = 0.25
_IOU_THRESH = 0.45
_MAX_PER_IMG = 100
_N = 5000
_NC = 80
_R = 40
_L = 128
_NP = _R * _L
_NEG = float("-inf")
_K = 4


_SX1, _SY1, _SX2, _SY2, _SOBJ, _SCCF, _SCPRED, _SAREA = range(8)


def _setup_one(x_ref, scr_ref, k):
    cx = x_ref[k, 0]
    cy = x_ref[k, 1]
    w = x_ref[k, 2]
    h = x_ref[k, 3]
    obj = x_ref[k, 4]

    x1 = cx - w / 2.0
    y1 = cy - h / 2.0
    x2 = cx + w / 2.0
    y2 = cy + h / 2.0
    area = (x2 - x1) * (y2 - y1)

    cls = x_ref[k, 5:5 + _NC]
    ccf = jnp.max(cls, axis=0)
    cidx = jax.lax.broadcasted_iota(jnp.int32, (_NC, _R, _L), 0)
    cpred = jnp.min(jnp.where(cls == ccf[None], cidx, 2**30), axis=0).astype(jnp.float32)

    score = obj * ccf
    ri = jax.lax.broadcasted_iota(jnp.int32, (_R, _L), 0)
    li = jax.lax.broadcasted_iota(jnp.int32, (_R, _L), 1)
    flat = ri * _L + li
    valid = flat < _N

    m0 = jnp.max(jnp.where(valid, score, _NEG), keepdims=True)
    conf = jnp.minimum(_CONF_THRESH, m0)
    s0 = jnp.where(valid & (score >= conf), score, _NEG)

    for cp in range(2):
        scr_ref[k, cp, _SX1] = x1
        scr_ref[k, cp, _SY1] = y1
        scr_ref[k, cp, _SX2] = x2
        scr_ref[k, cp, _SY2] = y2
        scr_ref[k, cp, _SOBJ] = obj
        scr_ref[k, cp, _SCCF] = ccf
        scr_ref[k, cp, _SCPRED] = cpred
        scr_ref[k, cp, _SAREA] = area
    return s0


def _nms_body(x_ref, o_ref, scr_ref):
    s0s = [_setup_one(x_ref, scr_ref, k) for k in range(_K)]
    lane8 = jax.lax.broadcasted_iota(jnp.int32, (1, 8), 1)

    def step(t, ss):
        cp = jax.lax.rem(t, 2)
        ri = jax.lax.broadcasted_iota(jnp.int32, (_R, _L), 0)
        li = jax.lax.broadcasted_iota(jnp.int32, (_R, _L), 1)
        flatf = (ri * _L + li).astype(jnp.float32)
        sub8 = jax.lax.broadcasted_iota(jnp.int32, (8, _L), 0)
        diag8 = (jax.lax.broadcasted_iota(jnp.int32, (8, 8), 0)
                 == jax.lax.broadcasted_iota(jnp.int32, (8, 8), 1))
        ones_mat = jnp.ones((_L, _L), jnp.float32)
        out = []
        rows = []
        for k in range(_K):
            s = ss[k]
            m = jnp.max(s, keepdims=True)
            i = jnp.min(jnp.where(s == m, flatf, 3.0e7), keepdims=True)
            pick = flatf == i
            pf = pick.astype(jnp.float32)

            sel = jnp.zeros((8, _L), jnp.float32)
            for j in range(8):
                colsum = jnp.sum(pf * scr_ref[k, cp, j], axis=0, keepdims=True)
                sel = sel + jnp.where(sub8 == j, colsum, 0.0)
            bvals = jnp.dot(sel, ones_mat, preferred_element_type=jnp.float32)
            bx1 = bvals[_SX1:_SX1 + 1, :]
            by1 = bvals[_SY1:_SY1 + 1, :]
            bx2 = bvals[_SX2:_SX2 + 1, :]
            by2 = bvals[_SY2:_SY2 + 1, :]
            bcls = bvals[_SCPRED:_SCPRED + 1, :]

            okf = jnp.where(m > _NEG, 1.0, 0.0)
            row = jnp.sum(jnp.where(diag8, bvals[:, :8], 0.0), axis=0,
                          keepdims=True) * okf
            rows.append(row)

            x1 = scr_ref[k, cp, _SX1]
            y1 = scr_ref[k, cp, _SY1]
            x2 = scr_ref[k, cp, _SX2]
            y2 = scr_ref[k, cp, _SY2]
            area = scr_ref[k, cp, _SAREA]
            cpred = scr_ref[k, cp, _SCPRED]
            xx1 = jnp.maximum(bx1, x1)
            yy1 = jnp.maximum(by1, y1)
            xx2 = jnp.minimum(bx2, x2)
            yy2 = jnp.minimum(by2, y2)
            inter = jnp.maximum(xx2 - xx1, 0.0) * jnp.maximum(yy2 - yy1, 0.0)
            ba = (bx2 - bx1) * (by2 - by1)
            iou = inter / (ba + area - inter + 1e-9)
            sup = (iou > _IOU_THRESH) & (cpred == bcls)
            out.append(jnp.where(sup | pick, _NEG, s))
        for k in range(_K):
            o_ref[k, pl.ds(t, 1), :] = rows[k]
        return tuple(out)

    jax.lax.fori_loop(0, _MAX_PER_IMG, step, tuple(s0s))


def kernel(x):
    b, n, c = x.shape
    xp = jnp.pad(x, ((0, 0), (0, _NP - n), (0, 0)))
    xt = xp.transpose(0, 2, 1).reshape(b, c, _R, _L)
    out = pl.pallas_call(
        _nms_body,
        grid=(b // _K,),
        in_specs=[pl.BlockSpec((_K, c, _R, _L), lambda i: (i, 0, 0, 0))],
        out_specs=pl.BlockSpec((_K, _MAX_PER_IMG, 8), lambda i: (i, 0, 0)),
        out_shape=jax.ShapeDtypeStruct((b, _MAX_PER_IMG, 8), jnp.float32),
        scratch_shapes=[pltpu.VMEM((_K, 2, 8, _R, _L), jnp.float32)],
        compiler_params=pltpu.CompilerParams(dimension_semantics=("parallel",)),
    )(xt)
    return out[:, :, :7]
